# baseline (speedup 1.0000x reference)
import jax
import jax.numpy as jnp
from jax.experimental import pallas as pl
from jax.experimental.pallas import tpu as pltpu

N = 10000
F = 512
H1 = 512
H2 = 256

BM = 1024
NP = 10240
NI = NP // BM
BKC = 1280
NKC = NP // BKC
BD = 2048
ND = NP // BD


def _row_mask(i, shape):
    rows = jax.lax.broadcasted_iota(jnp.int32, (shape[0], 1), 0) + i * shape[0]
    return rows < N


def _k1_body(x_ref, wh1_ref, wh2_ref, wam_ref, was_ref, epsa_ref,
             y_ref, zam_ref, zas_ref, zabf_ref, acc_ref):
    k = pl.program_id(0)
    valid = _row_mask(k, (BM, 1))
    x = jnp.where(valid, x_ref[...], 0.0).astype(jnp.bfloat16)
    y_ref[...] = jax.lax.dot_general(
        x, wh1_ref[...], (((1,), (0,)), ((), ())),
        preferred_element_type=jnp.float32).astype(jnp.bfloat16)
    w2 = jnp.where(valid, wh2_ref[...], 0.0).astype(jnp.bfloat16)
    part = jax.lax.dot_general(
        x, w2, (((0,), (0,)), ((), ())), preferred_element_type=jnp.float32)

    @pl.when(k == 0)
    def _():
        acc_ref[...] = part

    @pl.when(k > 0)
    def _():
        acc_ref[...] += part

    @pl.when(k == pl.num_programs(0) - 1)
    def _():
        za1 = jnp.tanh(acc_ref[...]).astype(jnp.bfloat16)
        zam = jax.lax.dot_general(
            za1, wam_ref[...], (((1,), (0,)), ((), ())),
            preferred_element_type=jnp.float32)
        zas = jax.lax.dot_general(
            za1, was_ref[...], (((1,), (0,)), ((), ())),
            preferred_element_type=jnp.float32)
        zam_ref[...] = zam
        zas_ref[...] = zas
        zabf_ref[...] = (zam + epsa_ref[...] * jnp.exp(zas)).astype(jnp.bfloat16)


def _k2_body(adj_ref, cmask_ref, y_ref, wcat_ref, m_ref, acc_ref):
    i = pl.program_id(0)
    k = pl.program_id(1)
    a_head = adj_ref[:, :BKC - 512].astype(jnp.bfloat16)
    a_tail = jnp.where(cmask_ref[0, :, BKC - 512:] > 0,
                       adj_ref[:, BKC - 512:].astype(jnp.bfloat16),
                       jnp.bfloat16(0.0))
    base = k * BKC
    part = jax.lax.dot_general(
        a_head, y_ref[pl.ds(base, BKC - 512), :], (((1,), (0,)), ((), ())),
        preferred_element_type=jnp.float32)
    part += jax.lax.dot_general(
        a_tail, y_ref[pl.ds(base + BKC - 512, 512), :], (((1,), (0,)), ((), ())),
        preferred_element_type=jnp.float32)

    @pl.when(k == 0)
    def _():
        acc_ref[...] = part

    @pl.when(k > 0)
    def _():
        acc_ref[...] += part

    @pl.when(k == pl.num_programs(1) - 1)
    def _():
        z1 = jnp.maximum(acc_ref[...], 0.0).astype(jnp.bfloat16)
        m = jax.lax.dot_general(
            z1, wcat_ref[...], (((1,), (0,)), ((), ())),
            preferred_element_type=jnp.float32)
        m_ref[...] = jnp.where(_row_mask(i, (BM, 1)), m, 0.0).astype(jnp.bfloat16)


def _k3_body(adj_ref, cmask_ref, m_ref, epsu_ref, zum_ref, zus_ref, zubf_ref,
             acc_ref):
    k = pl.program_id(1)
    a_head = adj_ref[:, :BKC - 512].astype(jnp.bfloat16)
    a_tail = jnp.where(cmask_ref[0, :, BKC - 512:] > 0,
                       adj_ref[:, BKC - 512:].astype(jnp.bfloat16),
                       jnp.bfloat16(0.0))
    base = k * BKC
    part = jax.lax.dot_general(
        a_head, m_ref[pl.ds(base, BKC - 512), :], (((1,), (0,)), ((), ())),
        preferred_element_type=jnp.float32)
    part += jax.lax.dot_general(
        a_tail, m_ref[pl.ds(base + BKC - 512, 512), :], (((1,), (0,)), ((), ())),
        preferred_element_type=jnp.float32)

    @pl.when(k == 0)
    def _():
        acc_ref[...] = part

    @pl.when(k > 0)
    def _():
        acc_ref[...] += part

    @pl.when(k == pl.num_programs(1) - 1)
    def _():
        u = acc_ref[...]
        zum = u[:, :H2]
        zus = u[:, H2:]
        zum_ref[...] = zum
        zus_ref[...] = zus
        zubf_ref[...] = (zum + epsu_ref[...] * jnp.exp(zus)).astype(jnp.bfloat16)


def _k4_body(zui_ref, zuj_ref, za_ref, pu_ref, pa_ref):
    j = pl.program_id(1)
    zui = zui_ref[...]
    pu_ref[...] = jax.lax.dot_general(
        zui, zuj_ref[...], (((1,), (1,)), ((), ())),
        preferred_element_type=jnp.float32)

    @pl.when(j == 0)
    def _():
        pa_ref[...] = jax.lax.dot_general(
            zui, za_ref[...], (((1,), (1,)), ((), ())),
            preferred_element_type=jnp.float32)


def kernel(features, adj, W_h1, W_h2, W_um, W_us, W_am, W_as, eps_u, eps_a):
    wh1 = W_h1.astype(jnp.bfloat16)
    wcat = jnp.concatenate([W_um, W_us], axis=1).astype(jnp.bfloat16)
    wam = W_am.astype(jnp.bfloat16)
    was = W_as.astype(jnp.bfloat16)
    cmask = (jnp.arange(NP, dtype=jnp.int32) < N).astype(
        jnp.float32).reshape(NKC, 1, BKC)

    y, za_mean, za_log_std, za_bf = pl.pallas_call(
        _k1_body,
        grid=(NI,),
        in_specs=[
            pl.BlockSpec((BM, F), lambda k: (k, 0)),
            pl.BlockSpec((F, H1), lambda k: (0, 0)),
            pl.BlockSpec((BM, H1), lambda k: (k, 0)),
            pl.BlockSpec((H1, H2), lambda k: (0, 0)),
            pl.BlockSpec((H1, H2), lambda k: (0, 0)),
            pl.BlockSpec((F, H2), lambda k: (0, 0)),
        ],
        out_specs=[
            pl.BlockSpec((BM, H1), lambda k: (k, 0)),
            pl.BlockSpec((F, H2), lambda k: (0, 0)),
            pl.BlockSpec((F, H2), lambda k: (0, 0)),
            pl.BlockSpec((F, H2), lambda k: (0, 0)),
        ],
        out_shape=[
            jax.ShapeDtypeStruct((NP, H1), jnp.bfloat16),
            jax.ShapeDtypeStruct((F, H2), jnp.float32),
            jax.ShapeDtypeStruct((F, H2), jnp.float32),
            jax.ShapeDtypeStruct((F, H2), jnp.bfloat16),
        ],
        scratch_shapes=[pltpu.VMEM((H1, H1), jnp.float32)],
    )(features, wh1, W_h2, wam, was, eps_a)

    m = pl.pallas_call(
        _k2_body,
        grid=(NI, NKC),
        in_specs=[
            pl.BlockSpec((BM, BKC), lambda i, k: (i, k)),
            pl.BlockSpec((1, 1, BKC), lambda i, k: (k, 0, 0)),
            pl.BlockSpec((NP, H1), lambda i, k: (0, 0)),
            pl.BlockSpec((H1, 2 * H2), lambda i, k: (0, 0)),
        ],
        out_specs=pl.BlockSpec((BM, 2 * H2), lambda i, k: (i, 0)),
        out_shape=jax.ShapeDtypeStruct((NP, 2 * H2), jnp.bfloat16),
        scratch_shapes=[pltpu.VMEM((BM, 2 * H2), jnp.float32)],
        compiler_params=pltpu.CompilerParams(
            dimension_semantics=("parallel", "arbitrary")),
    )(adj, cmask, y, wcat)

    zu_mean, zu_log_std, zu_bf = pl.pallas_call(
        _k3_body,
        grid=(NI, NKC),
        in_specs=[
            pl.BlockSpec((BM, BKC), lambda i, k: (i, k)),
            pl.BlockSpec((1, 1, BKC), lambda i, k: (k, 0, 0)),
            pl.BlockSpec((NP, 2 * H2), lambda i, k: (0, 0)),
            pl.BlockSpec((BM, H2), lambda i, k: (i, 0)),
        ],
        out_specs=[
            pl.BlockSpec((BM, H2), lambda i, k: (i, 0)),
            pl.BlockSpec((BM, H2), lambda i, k: (i, 0)),
            pl.BlockSpec((BM, H2), lambda i, k: (i, 0)),
        ],
        out_shape=[
            jax.ShapeDtypeStruct((N, H2), jnp.float32),
            jax.ShapeDtypeStruct((N, H2), jnp.float32),
            jax.ShapeDtypeStruct((N, H2), jnp.bfloat16),
        ],
        scratch_shapes=[pltpu.VMEM((BM, 2 * H2), jnp.float32)],
        compiler_params=pltpu.CompilerParams(
            dimension_semantics=("parallel", "arbitrary")),
    )(adj, cmask, m, eps_u)

    preds_u, preds_a = pl.pallas_call(
        _k4_body,
        grid=(ND, ND),
        in_specs=[
            pl.BlockSpec((BD, H2), lambda i, j: (i, 0)),
            pl.BlockSpec((BD, H2), lambda i, j: (j, 0)),
            pl.BlockSpec((F, H2), lambda i, j: (0, 0)),
        ],
        out_specs=[
            pl.BlockSpec((BD, BD), lambda i, j: (i, j)),
            pl.BlockSpec((BD, F), lambda i, j: (i, 0)),
        ],
        out_shape=[
            jax.ShapeDtypeStruct((N, N), jnp.float32),
            jax.ShapeDtypeStruct((N, F), jnp.float32),
        ],
        compiler_params=pltpu.CompilerParams(
            dimension_semantics=("parallel", "arbitrary")),
    )(zu_bf, zu_bf, za_bf)

    return (preds_u, preds_a, zu_mean, zu_log_std, za_mean, za_log_std)


# speedup vs baseline: 1.1335x; 1.1335x over previous
import jax
import jax.numpy as jnp
from jax.experimental import pallas as pl
from jax.experimental.pallas import tpu as pltpu

N = 10000
F = 512
H1 = 512
H2 = 256

BM = 1024
NP = 10240
NI = NP // BM
BKC = 2560
NKC = NP // BKC
BD = 2048
ND = NP // BD


def _row_mask(i, shape):
    rows = jax.lax.broadcasted_iota(jnp.int32, (shape[0], 1), 0) + i * shape[0]
    return rows < N


def _k1_body(x_ref, wh1_ref, wh2_ref, wam_ref, was_ref, epsa_ref,
             y_ref, zam_ref, zas_ref, zabf_ref, acc_ref):
    k = pl.program_id(0)
    valid = _row_mask(k, (BM, 1))
    x = jnp.where(valid, x_ref[...], 0.0).astype(jnp.bfloat16)
    y_ref[...] = jax.lax.dot_general(
        x, wh1_ref[...], (((1,), (0,)), ((), ())),
        preferred_element_type=jnp.float32).astype(jnp.bfloat16)
    w2 = jnp.where(valid, wh2_ref[...], 0.0).astype(jnp.bfloat16)
    part = jax.lax.dot_general(
        x, w2, (((0,), (0,)), ((), ())), preferred_element_type=jnp.float32)

    @pl.when(k == 0)
    def _():
        acc_ref[...] = part

    @pl.when(k > 0)
    def _():
        acc_ref[...] += part

    @pl.when(k == pl.num_programs(0) - 1)
    def _():
        za1 = jnp.tanh(acc_ref[...]).astype(jnp.bfloat16)
        zam = jax.lax.dot_general(
            za1, wam_ref[...], (((1,), (0,)), ((), ())),
            preferred_element_type=jnp.float32)
        zas = jax.lax.dot_general(
            za1, was_ref[...], (((1,), (0,)), ((), ())),
            preferred_element_type=jnp.float32)
        zam_ref[...] = zam
        zas_ref[...] = zas
        zabf_ref[...] = (zam + epsa_ref[...] * jnp.exp(zas)).astype(jnp.bfloat16)


def _k2_body(adj_ref, cmask_ref, y_ref, m_ref, acc_ref):
    i = pl.program_id(0)
    k = pl.program_id(1)
    a_head = adj_ref[:, :BKC - 512].astype(jnp.bfloat16)
    a_tail = jnp.where(cmask_ref[0, :, BKC - 512:] > 0,
                       adj_ref[:, BKC - 512:].astype(jnp.bfloat16),
                       jnp.bfloat16(0.0))
    base = k * BKC
    part = jax.lax.dot_general(
        a_head, y_ref[pl.ds(base, BKC - 512), :], (((1,), (0,)), ((), ())),
        preferred_element_type=jnp.float32)
    part += jax.lax.dot_general(
        a_tail, y_ref[pl.ds(base + BKC - 512, 512), :], (((1,), (0,)), ((), ())),
        preferred_element_type=jnp.float32)

    @pl.when(k == 0)
    def _():
        acc_ref[...] = part

    @pl.when(k > 0)
    def _():
        acc_ref[...] += part

    @pl.when(k == pl.num_programs(1) - 1)
    def _():
        z1 = jnp.maximum(acc_ref[...], 0.0)
        m_ref[...] = jnp.where(_row_mask(i, (BM, 1)), z1, 0.0).astype(jnp.bfloat16)


def _k3_body(adj_ref, cmask_ref, m_ref, wcat_ref, epsu_ref, zum_ref, zus_ref,
             zubf_ref, acc_ref):
    k = pl.program_id(1)
    a_head = adj_ref[:, :BKC - 512].astype(jnp.bfloat16)
    a_tail = jnp.where(cmask_ref[0, :, BKC - 512:] > 0,
                       adj_ref[:, BKC - 512:].astype(jnp.bfloat16),
                       jnp.bfloat16(0.0))
    base = k * BKC
    part = jax.lax.dot_general(
        a_head, m_ref[pl.ds(base, BKC - 512), :], (((1,), (0,)), ((), ())),
        preferred_element_type=jnp.float32)
    part += jax.lax.dot_general(
        a_tail, m_ref[pl.ds(base + BKC - 512, 512), :], (((1,), (0,)), ((), ())),
        preferred_element_type=jnp.float32)

    @pl.when(k == 0)
    def _():
        acc_ref[...] = part

    @pl.when(k > 0)
    def _():
        acc_ref[...] += part

    @pl.when(k == pl.num_programs(1) - 1)
    def _():
        u = jax.lax.dot_general(
            acc_ref[...].astype(jnp.bfloat16), wcat_ref[...],
            (((1,), (0,)), ((), ())), preferred_element_type=jnp.float32)
        zum = u[:, :H2]
        zus = u[:, H2:]
        zum_ref[...] = zum
        zus_ref[...] = zus
        zubf_ref[...] = (zum + epsu_ref[...] * jnp.exp(zus)).astype(jnp.bfloat16)


def _k4_body(zui_ref, zuj_ref, za_ref, pu_ref, pa_ref):
    j = pl.program_id(1)
    zui = zui_ref[...]
    pu_ref[...] = jax.lax.dot_general(
        zui, zuj_ref[...], (((1,), (1,)), ((), ())),
        preferred_element_type=jnp.float32)

    @pl.when(j == 0)
    def _():
        pa_ref[...] = jax.lax.dot_general(
            zui, za_ref[...], (((1,), (1,)), ((), ())),
            preferred_element_type=jnp.float32)


def kernel(features, adj, W_h1, W_h2, W_um, W_us, W_am, W_as, eps_u, eps_a):
    wh1 = W_h1.astype(jnp.bfloat16)
    wcat = jnp.concatenate([W_um, W_us], axis=1).astype(jnp.bfloat16)
    wam = W_am.astype(jnp.bfloat16)
    was = W_as.astype(jnp.bfloat16)
    cmask = (jnp.arange(NP, dtype=jnp.int32) < N).astype(
        jnp.float32).reshape(NKC, 1, BKC)

    y, za_mean, za_log_std, za_bf = pl.pallas_call(
        _k1_body,
        grid=(NI,),
        in_specs=[
            pl.BlockSpec((BM, F), lambda k: (k, 0)),
            pl.BlockSpec((F, H1), lambda k: (0, 0)),
            pl.BlockSpec((BM, H1), lambda k: (k, 0)),
            pl.BlockSpec((H1, H2), lambda k: (0, 0)),
            pl.BlockSpec((H1, H2), lambda k: (0, 0)),
            pl.BlockSpec((F, H2), lambda k: (0, 0)),
        ],
        out_specs=[
            pl.BlockSpec((BM, H1), lambda k: (k, 0)),
            pl.BlockSpec((F, H2), lambda k: (0, 0)),
            pl.BlockSpec((F, H2), lambda k: (0, 0)),
            pl.BlockSpec((F, H2), lambda k: (0, 0)),
        ],
        out_shape=[
            jax.ShapeDtypeStruct((NP, H1), jnp.bfloat16),
            jax.ShapeDtypeStruct((F, H2), jnp.float32),
            jax.ShapeDtypeStruct((F, H2), jnp.float32),
            jax.ShapeDtypeStruct((F, H2), jnp.bfloat16),
        ],
        scratch_shapes=[pltpu.VMEM((H1, H1), jnp.float32)],
    )(features, wh1, W_h2, wam, was, eps_a)

    m = pl.pallas_call(
        _k2_body,
        grid=(NI, NKC),
        in_specs=[
            pl.BlockSpec((BM, BKC), lambda i, k: (i, k)),
            pl.BlockSpec((1, 1, BKC), lambda i, k: (k, 0, 0)),
            pl.BlockSpec((NP, H1), lambda i, k: (0, 0)),
        ],
        out_specs=pl.BlockSpec((BM, H1), lambda i, k: (i, 0)),
        out_shape=jax.ShapeDtypeStruct((NP, H1), jnp.bfloat16),
        scratch_shapes=[pltpu.VMEM((BM, H1), jnp.float32)],
        compiler_params=pltpu.CompilerParams(
            dimension_semantics=("parallel", "arbitrary")),
    )(adj, cmask, y)

    zu_mean, zu_log_std, zu_bf = pl.pallas_call(
        _k3_body,
        grid=(NI, NKC),
        in_specs=[
            pl.BlockSpec((BM, BKC), lambda i, k: (i, k)),
            pl.BlockSpec((1, 1, BKC), lambda i, k: (k, 0, 0)),
            pl.BlockSpec((NP, H1), lambda i, k: (0, 0)),
            pl.BlockSpec((H1, 2 * H2), lambda i, k: (0, 0)),
            pl.BlockSpec((BM, H2), lambda i, k: (i, 0)),
        ],
        out_specs=[
            pl.BlockSpec((BM, H2), lambda i, k: (i, 0)),
            pl.BlockSpec((BM, H2), lambda i, k: (i, 0)),
            pl.BlockSpec((BM, H2), lambda i, k: (i, 0)),
        ],
        out_shape=[
            jax.ShapeDtypeStruct((N, H2), jnp.float32),
            jax.ShapeDtypeStruct((N, H2), jnp.float32),
            jax.ShapeDtypeStruct((N, H2), jnp.bfloat16),
        ],
        scratch_shapes=[pltpu.VMEM((BM, 2 * H2), jnp.float32)],
        compiler_params=pltpu.CompilerParams(
            dimension_semantics=("parallel", "arbitrary")),
    )(adj, cmask, m, wcat, eps_u)

    preds_u, preds_a = pl.pallas_call(
        _k4_body,
        grid=(ND, ND),
        in_specs=[
            pl.BlockSpec((BD, H2), lambda i, j: (i, 0)),
            pl.BlockSpec((BD, H2), lambda i, j: (j, 0)),
            pl.BlockSpec((F, H2), lambda i, j: (0, 0)),
        ],
        out_specs=[
            pl.BlockSpec((BD, BD), lambda i, j: (i, j)),
            pl.BlockSpec((BD, F), lambda i, j: (i, 0)),
        ],
        out_shape=[
            jax.ShapeDtypeStruct((N, N), jnp.float32),
            jax.ShapeDtypeStruct((N, F), jnp.float32),
        ],
        compiler_params=pltpu.CompilerParams(
            dimension_semantics=("parallel", "arbitrary")),
    )(zu_bf, zu_bf, za_bf)

    return (preds_u, preds_a, zu_mean, zu_log_std, za_mean, za_log_std)


# speedup vs baseline: 1.1644x; 1.0273x over previous
import jax
import jax.numpy as jnp
from jax.experimental import pallas as pl
from jax.experimental.pallas import tpu as pltpu

N = 10000
F = 512
H1 = 512
H2 = 256

BM = 1024
NP = 10240
NI = NP // BM
BKC = 5120
NKC = NP // BKC
BM3 = 512
BD = 2048
ND = NP // BD


def _row_mask(i, shape):
    rows = jax.lax.broadcasted_iota(jnp.int32, (shape[0], 1), 0) + i * shape[0]
    return rows < N


def _k1_body(x_ref, wh1_ref, wh2_ref, wam_ref, was_ref, epsa_ref,
             y_ref, zam_ref, zas_ref, zabf_ref, acc_ref):
    k = pl.program_id(0)
    valid = _row_mask(k, (BM, 1))
    x = jnp.where(valid, x_ref[...], 0.0).astype(jnp.bfloat16)
    y_ref[...] = jax.lax.dot_general(
        x, wh1_ref[...], (((1,), (0,)), ((), ())),
        preferred_element_type=jnp.float32).astype(jnp.bfloat16)
    w2 = jnp.where(valid, wh2_ref[...], 0.0).astype(jnp.bfloat16)
    part = jax.lax.dot_general(
        x, w2, (((0,), (0,)), ((), ())), preferred_element_type=jnp.float32)

    @pl.when(k == 0)
    def _():
        acc_ref[...] = part

    @pl.when(k > 0)
    def _():
        acc_ref[...] += part

    @pl.when(k == pl.num_programs(0) - 1)
    def _():
        za1 = jnp.tanh(acc_ref[...]).astype(jnp.bfloat16)
        zam = jax.lax.dot_general(
            za1, wam_ref[...], (((1,), (0,)), ((), ())),
            preferred_element_type=jnp.float32)
        zas = jax.lax.dot_general(
            za1, was_ref[...], (((1,), (0,)), ((), ())),
            preferred_element_type=jnp.float32)
        zam_ref[...] = zam
        zas_ref[...] = zas
        zabf_ref[...] = (zam + epsa_ref[...] * jnp.exp(zas)).astype(jnp.bfloat16)


def _k2_body(adj_ref, cmask_ref, y_ref, m_ref, acc_ref):
    i = pl.program_id(0)
    k = pl.program_id(1)
    a_head = adj_ref[:, :BKC - 512].astype(jnp.bfloat16)
    a_tail = jnp.where(cmask_ref[0, :, BKC - 512:] > 0,
                       adj_ref[:, BKC - 512:].astype(jnp.bfloat16),
                       jnp.bfloat16(0.0))
    base = k * BKC
    part = jax.lax.dot_general(
        a_head, y_ref[pl.ds(base, BKC - 512), :], (((1,), (0,)), ((), ())),
        preferred_element_type=jnp.float32)
    part += jax.lax.dot_general(
        a_tail, y_ref[pl.ds(base + BKC - 512, 512), :], (((1,), (0,)), ((), ())),
        preferred_element_type=jnp.float32)

    @pl.when(k == 0)
    def _():
        acc_ref[...] = part

    @pl.when(k > 0)
    def _():
        acc_ref[...] += part

    @pl.when(k == pl.num_programs(1) - 1)
    def _():
        z1 = jnp.maximum(acc_ref[...], 0.0)
        m_ref[...] = jnp.where(_row_mask(i, (BM, 1)), z1, 0.0).astype(jnp.bfloat16)


def _k3_body(adj_ref, cmask_ref, m_ref, wcat_ref, epsu_ref, zum_ref, zus_ref,
             zubf_ref, acc_ref):
    k = pl.program_id(1)
    a_head = adj_ref[:, :BKC - 512].astype(jnp.bfloat16)
    a_tail = jnp.where(cmask_ref[0, :, BKC - 512:] > 0,
                       adj_ref[:, BKC - 512:].astype(jnp.bfloat16),
                       jnp.bfloat16(0.0))
    base = k * BKC
    part = jax.lax.dot_general(
        a_head, m_ref[pl.ds(base, BKC - 512), :], (((1,), (0,)), ((), ())),
        preferred_element_type=jnp.float32)
    part += jax.lax.dot_general(
        a_tail, m_ref[pl.ds(base + BKC - 512, 512), :], (((1,), (0,)), ((), ())),
        preferred_element_type=jnp.float32)

    @pl.when(k == 0)
    def _():
        acc_ref[...] = part

    @pl.when(k > 0)
    def _():
        acc_ref[...] += part

    @pl.when(k == pl.num_programs(1) - 1)
    def _():
        u = jax.lax.dot_general(
            acc_ref[...].astype(jnp.bfloat16), wcat_ref[...],
            (((1,), (0,)), ((), ())), preferred_element_type=jnp.float32)
        zum = u[:, :H2]
        zus = u[:, H2:]
        zum_ref[...] = zum
        zus_ref[...] = zus
        zubf_ref[...] = (zum + epsu_ref[...] * jnp.exp(zus)).astype(jnp.bfloat16)


def _k4_body(zui_ref, zuj_ref, za_ref, pu_ref, pa_ref):
    j = pl.program_id(1)
    zui = zui_ref[...]
    pu_ref[...] = jax.lax.dot_general(
        zui, zuj_ref[...], (((1,), (1,)), ((), ())),
        preferred_element_type=jnp.float32)

    @pl.when(j == 0)
    def _():
        pa_ref[...] = jax.lax.dot_general(
            zui, za_ref[...], (((1,), (1,)), ((), ())),
            preferred_element_type=jnp.float32)


def kernel(features, adj, W_h1, W_h2, W_um, W_us, W_am, W_as, eps_u, eps_a):
    wh1 = W_h1.astype(jnp.bfloat16)
    wcat = jnp.concatenate([W_um, W_us], axis=1).astype(jnp.bfloat16)
    wam = W_am.astype(jnp.bfloat16)
    was = W_as.astype(jnp.bfloat16)
    cmask = (jnp.arange(NP, dtype=jnp.int32) < N).astype(
        jnp.float32).reshape(NKC, 1, BKC)

    y, za_mean, za_log_std, za_bf = pl.pallas_call(
        _k1_body,
        grid=(NI,),
        in_specs=[
            pl.BlockSpec((BM, F), lambda k: (k, 0)),
            pl.BlockSpec((F, H1), lambda k: (0, 0)),
            pl.BlockSpec((BM, H1), lambda k: (k, 0)),
            pl.BlockSpec((H1, H2), lambda k: (0, 0)),
            pl.BlockSpec((H1, H2), lambda k: (0, 0)),
            pl.BlockSpec((F, H2), lambda k: (0, 0)),
        ],
        out_specs=[
            pl.BlockSpec((BM, H1), lambda k: (k, 0)),
            pl.BlockSpec((F, H2), lambda k: (0, 0)),
            pl.BlockSpec((F, H2), lambda k: (0, 0)),
            pl.BlockSpec((F, H2), lambda k: (0, 0)),
        ],
        out_shape=[
            jax.ShapeDtypeStruct((NP, H1), jnp.bfloat16),
            jax.ShapeDtypeStruct((F, H2), jnp.float32),
            jax.ShapeDtypeStruct((F, H2), jnp.float32),
            jax.ShapeDtypeStruct((F, H2), jnp.bfloat16),
        ],
        scratch_shapes=[pltpu.VMEM((H1, H1), jnp.float32)],
    )(features, wh1, W_h2, wam, was, eps_a)

    m = pl.pallas_call(
        _k2_body,
        grid=(NI, NKC),
        in_specs=[
            pl.BlockSpec((BM, BKC), lambda i, k: (i, k)),
            pl.BlockSpec((1, 1, BKC), lambda i, k: (k, 0, 0)),
            pl.BlockSpec((NP, H1), lambda i, k: (0, 0)),
        ],
        out_specs=pl.BlockSpec((BM, H1), lambda i, k: (i, 0)),
        out_shape=jax.ShapeDtypeStruct((NP, H1), jnp.bfloat16),
        scratch_shapes=[pltpu.VMEM((BM, H1), jnp.float32)],
        compiler_params=pltpu.CompilerParams(
            dimension_semantics=("parallel", "arbitrary")),
    )(adj, cmask, y)

    zu_mean, zu_log_std, zu_bf = pl.pallas_call(
        _k3_body,
        grid=(NP // BM3, NKC),
        in_specs=[
            pl.BlockSpec((BM3, BKC), lambda i, k: (i, k)),
            pl.BlockSpec((1, 1, BKC), lambda i, k: (k, 0, 0)),
            pl.BlockSpec((NP, H1), lambda i, k: (0, 0)),
            pl.BlockSpec((H1, 2 * H2), lambda i, k: (0, 0)),
            pl.BlockSpec((BM3, H2), lambda i, k: (i, 0)),
        ],
        out_specs=[
            pl.BlockSpec((BM3, H2), lambda i, k: (i, 0)),
            pl.BlockSpec((BM3, H2), lambda i, k: (i, 0)),
            pl.BlockSpec((BM3, H2), lambda i, k: (i, 0)),
        ],
        out_shape=[
            jax.ShapeDtypeStruct((N, H2), jnp.float32),
            jax.ShapeDtypeStruct((N, H2), jnp.float32),
            jax.ShapeDtypeStruct((N, H2), jnp.bfloat16),
        ],
        scratch_shapes=[pltpu.VMEM((BM3, 2 * H2), jnp.float32)],
        compiler_params=pltpu.CompilerParams(
            dimension_semantics=("parallel", "arbitrary")),
    )(adj, cmask, m, wcat, eps_u)

    preds_u, preds_a = pl.pallas_call(
        _k4_body,
        grid=(ND, ND),
        in_specs=[
            pl.BlockSpec((BD, H2), lambda i, j: (i, 0)),
            pl.BlockSpec((BD, H2), lambda i, j: (j, 0)),
            pl.BlockSpec((F, H2), lambda i, j: (0, 0)),
        ],
        out_specs=[
            pl.BlockSpec((BD, BD), lambda i, j: (i, j)),
            pl.BlockSpec((BD, F), lambda i, j: (i, 0)),
        ],
        out_shape=[
            jax.ShapeDtypeStruct((N, N), jnp.float32),
            jax.ShapeDtypeStruct((N, F), jnp.float32),
        ],
        compiler_params=pltpu.CompilerParams(
            dimension_semantics=("parallel", "arbitrary")),
    )(zu_bf, zu_bf, za_bf)

    return (preds_u, preds_a, zu_mean, zu_log_std, za_mean, za_log_std)


# speedup vs baseline: 1.2205x; 1.0482x over previous
import jax
import jax.numpy as jnp
from jax.experimental import pallas as pl
from jax.experimental.pallas import tpu as pltpu

N = 10000
F = 512
H1 = 512
H2 = 256

BM = 1024
NP = 10240
NI = NP // BM
BMA = 512
NIA = NP // BMA
BD = 2048
ND = NP // BD


def _row_mask(i, shape):
    rows = jax.lax.broadcasted_iota(jnp.int32, (shape[0], 1), 0) + i * shape[0]
    return rows < N


def _k1_body(x_ref, wh1_ref, wh2_ref, wam_ref, was_ref, epsa_ref,
             y_ref, zam_ref, zas_ref, zabf_ref, acc_ref):
    k = pl.program_id(0)
    valid = _row_mask(k, (BM, 1))
    x = jnp.where(valid, x_ref[...], 0.0).astype(jnp.bfloat16)
    y_ref[...] = jax.lax.dot_general(
        x, wh1_ref[...], (((1,), (0,)), ((), ())),
        preferred_element_type=jnp.float32).astype(jnp.bfloat16)
    w2 = jnp.where(valid, wh2_ref[...], 0.0).astype(jnp.bfloat16)
    part = jax.lax.dot_general(
        x, w2, (((0,), (0,)), ((), ())), preferred_element_type=jnp.float32)

    @pl.when(k == 0)
    def _():
        acc_ref[...] = part

    @pl.when(k > 0)
    def _():
        acc_ref[...] += part

    @pl.when(k == pl.num_programs(0) - 1)
    def _():
        za1 = jnp.tanh(acc_ref[...]).astype(jnp.bfloat16)
        zam = jax.lax.dot_general(
            za1, wam_ref[...], (((1,), (0,)), ((), ())),
            preferred_element_type=jnp.float32)
        zas = jax.lax.dot_general(
            za1, was_ref[...], (((1,), (0,)), ((), ())),
            preferred_element_type=jnp.float32)
        zam_ref[...] = zam
        zas_ref[...] = zas
        zabf_ref[...] = (zam + epsa_ref[...] * jnp.exp(zas)).astype(jnp.bfloat16)


def _k2_body(adj_ref, cmask_ref, y_ref, m_ref):
    i = pl.program_id(0)
    a_head = adj_ref[:, :NP - 512].astype(jnp.bfloat16)
    a_tail = jnp.where(cmask_ref[0, :, NP - 512:] > 0,
                       adj_ref[:, NP - 512:].astype(jnp.bfloat16),
                       jnp.bfloat16(0.0))
    part = jax.lax.dot_general(
        a_head, y_ref[:NP - 512, :], (((1,), (0,)), ((), ())),
        preferred_element_type=jnp.float32)
    part += jax.lax.dot_general(
        a_tail, y_ref[NP - 512:, :], (((1,), (0,)), ((), ())),
        preferred_element_type=jnp.float32)
    z1 = jnp.maximum(part, 0.0)
    m_ref[...] = jnp.where(_row_mask(i, (BMA, 1)), z1, 0.0).astype(jnp.bfloat16)


def _k3_body(adj_ref, cmask_ref, m_ref, wcat_ref, epsu_ref, zum_ref, zus_ref,
             zubf_ref):
    a_head = adj_ref[:, :NP - 512].astype(jnp.bfloat16)
    a_tail = jnp.where(cmask_ref[0, :, NP - 512:] > 0,
                       adj_ref[:, NP - 512:].astype(jnp.bfloat16),
                       jnp.bfloat16(0.0))
    part = jax.lax.dot_general(
        a_head, m_ref[:NP - 512, :], (((1,), (0,)), ((), ())),
        preferred_element_type=jnp.float32)
    part += jax.lax.dot_general(
        a_tail, m_ref[NP - 512:, :], (((1,), (0,)), ((), ())),
        preferred_element_type=jnp.float32)
    u = jax.lax.dot_general(
        part.astype(jnp.bfloat16), wcat_ref[...],
        (((1,), (0,)), ((), ())), preferred_element_type=jnp.float32)
    zum = u[:, :H2]
    zus = u[:, H2:]
    zum_ref[...] = zum
    zus_ref[...] = zus
    zubf_ref[...] = (zum + epsu_ref[...] * jnp.exp(zus)).astype(jnp.bfloat16)


def _k4_body(zui_ref, zuj_ref, za_ref, pu_ref, pa_ref):
    j = pl.program_id(1)
    zui = zui_ref[...]
    pu_ref[...] = jax.lax.dot_general(
        zui, zuj_ref[...], (((1,), (1,)), ((), ())),
        preferred_element_type=jnp.float32)

    @pl.when(j == 0)
    def _():
        pa_ref[...] = jax.lax.dot_general(
            zui, za_ref[...], (((1,), (1,)), ((), ())),
            preferred_element_type=jnp.float32)


def kernel(features, adj, W_h1, W_h2, W_um, W_us, W_am, W_as, eps_u, eps_a):
    wh1 = W_h1.astype(jnp.bfloat16)
    wcat = jnp.concatenate([W_um, W_us], axis=1).astype(jnp.bfloat16)
    wam = W_am.astype(jnp.bfloat16)
    was = W_as.astype(jnp.bfloat16)
    cmask = (jnp.arange(NP, dtype=jnp.int32) < N).astype(
        jnp.float32).reshape(1, 1, NP)

    y, za_mean, za_log_std, za_bf = pl.pallas_call(
        _k1_body,
        grid=(NI,),
        in_specs=[
            pl.BlockSpec((BM, F), lambda k: (k, 0)),
            pl.BlockSpec((F, H1), lambda k: (0, 0)),
            pl.BlockSpec((BM, H1), lambda k: (k, 0)),
            pl.BlockSpec((H1, H2), lambda k: (0, 0)),
            pl.BlockSpec((H1, H2), lambda k: (0, 0)),
            pl.BlockSpec((F, H2), lambda k: (0, 0)),
        ],
        out_specs=[
            pl.BlockSpec((BM, H1), lambda k: (k, 0)),
            pl.BlockSpec((F, H2), lambda k: (0, 0)),
            pl.BlockSpec((F, H2), lambda k: (0, 0)),
            pl.BlockSpec((F, H2), lambda k: (0, 0)),
        ],
        out_shape=[
            jax.ShapeDtypeStruct((NP, H1), jnp.bfloat16),
            jax.ShapeDtypeStruct((F, H2), jnp.float32),
            jax.ShapeDtypeStruct((F, H2), jnp.float32),
            jax.ShapeDtypeStruct((F, H2), jnp.bfloat16),
        ],
        scratch_shapes=[pltpu.VMEM((H1, H1), jnp.float32)],
    )(features, wh1, W_h2, wam, was, eps_a)

    m = pl.pallas_call(
        _k2_body,
        grid=(NIA,),
        in_specs=[
            pl.BlockSpec((BMA, NP), lambda i: (i, 0)),
            pl.BlockSpec((1, 1, NP), lambda i: (0, 0, 0)),
            pl.BlockSpec((NP, H1), lambda i: (0, 0)),
        ],
        out_specs=pl.BlockSpec((BMA, H1), lambda i: (i, 0)),
        out_shape=jax.ShapeDtypeStruct((NP, H1), jnp.bfloat16),
        compiler_params=pltpu.CompilerParams(
            dimension_semantics=("arbitrary",)),
    )(adj, cmask, y)

    zu_mean, zu_log_std, zu_bf = pl.pallas_call(
        _k3_body,
        grid=(NIA,),
        in_specs=[
            pl.BlockSpec((BMA, NP), lambda i: (i, 0)),
            pl.BlockSpec((1, 1, NP), lambda i: (0, 0, 0)),
            pl.BlockSpec((NP, H1), lambda i: (0, 0)),
            pl.BlockSpec((H1, 2 * H2), lambda i: (0, 0)),
            pl.BlockSpec((BMA, H2), lambda i: (i, 0)),
        ],
        out_specs=[
            pl.BlockSpec((BMA, H2), lambda i: (i, 0)),
            pl.BlockSpec((BMA, H2), lambda i: (i, 0)),
            pl.BlockSpec((BMA, H2), lambda i: (i, 0)),
        ],
        out_shape=[
            jax.ShapeDtypeStruct((N, H2), jnp.float32),
            jax.ShapeDtypeStruct((N, H2), jnp.float32),
            jax.ShapeDtypeStruct((N, H2), jnp.bfloat16),
        ],
        compiler_params=pltpu.CompilerParams(
            dimension_semantics=("arbitrary",)),
    )(adj, cmask, m, wcat, eps_u)

    preds_u, preds_a = pl.pallas_call(
        _k4_body,
        grid=(ND, ND),
        in_specs=[
            pl.BlockSpec((BD, H2), lambda i, j: (i, 0)),
            pl.BlockSpec((BD, H2), lambda i, j: (j, 0)),
            pl.BlockSpec((F, H2), lambda i, j: (0, 0)),
        ],
        out_specs=[
            pl.BlockSpec((BD, BD), lambda i, j: (i, j)),
            pl.BlockSpec((BD, F), lambda i, j: (i, 0)),
        ],
        out_shape=[
            jax.ShapeDtypeStruct((N, N), jnp.float32),
            jax.ShapeDtypeStruct((N, F), jnp.float32),
        ],
        compiler_params=pltpu.CompilerParams(
            dimension_semantics=("parallel", "arbitrary")),
    )(zu_bf, zu_bf, za_bf)

    return (preds_u, preds_a, zu_mean, zu_log_std, za_mean, za_log_std)


# speedup vs baseline: 1.2326x; 1.0099x over previous
import jax
import jax.numpy as jnp
from jax.experimental import pallas as pl
from jax.experimental.pallas import tpu as pltpu

N = 10000
F = 512
H1 = 512
H2 = 256

BM = 1024
NP = 10240
NI = NP // BM
BMA = 512
NIA = NP // BMA
BD = 2048
ND = NP // BD


def _row_mask(i, shape):
    rows = jax.lax.broadcasted_iota(jnp.int32, (shape[0], 1), 0) + i * shape[0]
    return rows < N


def _k1_body(x_ref, wh1_ref, wh2_ref, wam_ref, was_ref, wum_ref, wus_ref,
             epsa_ref, y_ref, zam_ref, zas_ref, zabf_ref, wcat_ref, acc_ref):
    k = pl.program_id(0)
    valid = _row_mask(k, (BM, 1))
    x = jnp.where(valid, x_ref[...], 0.0).astype(jnp.bfloat16)
    y_ref[...] = jax.lax.dot_general(
        x, wh1_ref[...], (((1,), (0,)), ((), ())),
        preferred_element_type=jnp.float32).astype(jnp.bfloat16)
    w2 = jnp.where(valid, wh2_ref[...], 0.0).astype(jnp.bfloat16)
    part = jax.lax.dot_general(
        x, w2, (((0,), (0,)), ((), ())), preferred_element_type=jnp.float32)

    @pl.when(k == 0)
    def _():
        acc_ref[...] = part

    @pl.when(k > 0)
    def _():
        acc_ref[...] += part

    @pl.when(k == pl.num_programs(0) - 1)
    def _():
        za1 = jnp.tanh(acc_ref[...]).astype(jnp.bfloat16)
        zam = jax.lax.dot_general(
            za1, wam_ref[...], (((1,), (0,)), ((), ())),
            preferred_element_type=jnp.float32)
        zas = jax.lax.dot_general(
            za1, was_ref[...], (((1,), (0,)), ((), ())),
            preferred_element_type=jnp.float32)
        zam_ref[...] = zam
        zas_ref[...] = zas
        zabf_ref[...] = (zam + epsa_ref[...] * jnp.exp(zas)).astype(jnp.bfloat16)
        wcat_ref[:, :H2] = wum_ref[...].astype(jnp.bfloat16)
        wcat_ref[:, H2:] = wus_ref[...].astype(jnp.bfloat16)


def _tail_mask():
    cols = jax.lax.broadcasted_iota(jnp.int32, (1, 512), 1) + (NP - 512)
    return cols < N


def _k2_body(adj_ref, y_ref, m_ref):
    i = pl.program_id(0)
    a_head = adj_ref[:, :NP - 512].astype(jnp.bfloat16)
    a_tail = jnp.where(_tail_mask(), adj_ref[:, NP - 512:].astype(jnp.bfloat16),
                       jnp.bfloat16(0.0))
    part = jax.lax.dot_general(
        a_head, y_ref[:NP - 512, :], (((1,), (0,)), ((), ())),
        preferred_element_type=jnp.float32)
    part += jax.lax.dot_general(
        a_tail, y_ref[NP - 512:, :], (((1,), (0,)), ((), ())),
        preferred_element_type=jnp.float32)
    z1 = jnp.maximum(part, 0.0)
    m_ref[...] = jnp.where(_row_mask(i, (BMA, 1)), z1, 0.0).astype(jnp.bfloat16)


def _k3_body(adj_ref, m_ref, wcat_ref, epsu_ref, zum_ref, zus_ref,
             zubf_ref):
    a_head = adj_ref[:, :NP - 512].astype(jnp.bfloat16)
    a_tail = jnp.where(_tail_mask(), adj_ref[:, NP - 512:].astype(jnp.bfloat16),
                       jnp.bfloat16(0.0))
    part = jax.lax.dot_general(
        a_head, m_ref[:NP - 512, :], (((1,), (0,)), ((), ())),
        preferred_element_type=jnp.float32)
    part += jax.lax.dot_general(
        a_tail, m_ref[NP - 512:, :], (((1,), (0,)), ((), ())),
        preferred_element_type=jnp.float32)
    u = jax.lax.dot_general(
        part.astype(jnp.bfloat16), wcat_ref[...],
        (((1,), (0,)), ((), ())), preferred_element_type=jnp.float32)
    zum = u[:, :H2]
    zus = u[:, H2:]
    zum_ref[...] = zum
    zus_ref[...] = zus
    zubf_ref[...] = (zum + epsu_ref[...] * jnp.exp(zus)).astype(jnp.bfloat16)


def _k4_body(zui_ref, zuj_ref, za_ref, pu_ref, pa_ref):
    j = pl.program_id(1)
    zui = zui_ref[...]
    pu_ref[...] = jax.lax.dot_general(
        zui, zuj_ref[...], (((1,), (1,)), ((), ())),
        preferred_element_type=jnp.float32)

    @pl.when(j == 0)
    def _():
        pa_ref[...] = jax.lax.dot_general(
            zui, za_ref[...], (((1,), (1,)), ((), ())),
            preferred_element_type=jnp.float32)


def kernel(features, adj, W_h1, W_h2, W_um, W_us, W_am, W_as, eps_u, eps_a):
    wh1 = W_h1.astype(jnp.bfloat16)
    wam = W_am.astype(jnp.bfloat16)
    was = W_as.astype(jnp.bfloat16)

    y, za_mean, za_log_std, za_bf, wcat = pl.pallas_call(
        _k1_body,
        grid=(NI,),
        in_specs=[
            pl.BlockSpec((BM, F), lambda k: (k, 0)),
            pl.BlockSpec((F, H1), lambda k: (0, 0)),
            pl.BlockSpec((BM, H1), lambda k: (k, 0)),
            pl.BlockSpec((H1, H2), lambda k: (0, 0)),
            pl.BlockSpec((H1, H2), lambda k: (0, 0)),
            pl.BlockSpec((H1, H2), lambda k: (0, 0)),
            pl.BlockSpec((H1, H2), lambda k: (0, 0)),
            pl.BlockSpec((F, H2), lambda k: (0, 0)),
        ],
        out_specs=[
            pl.BlockSpec((BM, H1), lambda k: (k, 0)),
            pl.BlockSpec((F, H2), lambda k: (0, 0)),
            pl.BlockSpec((F, H2), lambda k: (0, 0)),
            pl.BlockSpec((F, H2), lambda k: (0, 0)),
            pl.BlockSpec((H1, 2 * H2), lambda k: (0, 0)),
        ],
        out_shape=[
            jax.ShapeDtypeStruct((NP, H1), jnp.bfloat16),
            jax.ShapeDtypeStruct((F, H2), jnp.float32),
            jax.ShapeDtypeStruct((F, H2), jnp.float32),
            jax.ShapeDtypeStruct((F, H2), jnp.bfloat16),
            jax.ShapeDtypeStruct((H1, 2 * H2), jnp.bfloat16),
        ],
        scratch_shapes=[pltpu.VMEM((H1, H1), jnp.float32)],
    )(features, wh1, W_h2, wam, was, W_um, W_us, eps_a)

    m = pl.pallas_call(
        _k2_body,
        grid=(NIA,),
        in_specs=[
            pl.BlockSpec((BMA, NP), lambda i: (i, 0)),
            pl.BlockSpec((NP, H1), lambda i: (0, 0)),
        ],
        out_specs=pl.BlockSpec((BMA, H1), lambda i: (i, 0)),
        out_shape=jax.ShapeDtypeStruct((NP, H1), jnp.bfloat16),
        compiler_params=pltpu.CompilerParams(
            dimension_semantics=("arbitrary",)),
    )(adj, y)

    zu_mean, zu_log_std, zu_bf = pl.pallas_call(
        _k3_body,
        grid=(NIA,),
        in_specs=[
            pl.BlockSpec((BMA, NP), lambda i: (i, 0)),
            pl.BlockSpec((NP, H1), lambda i: (0, 0)),
            pl.BlockSpec((H1, 2 * H2), lambda i: (0, 0)),
            pl.BlockSpec((BMA, H2), lambda i: (i, 0)),
        ],
        out_specs=[
            pl.BlockSpec((BMA, H2), lambda i: (i, 0)),
            pl.BlockSpec((BMA, H2), lambda i: (i, 0)),
            pl.BlockSpec((BMA, H2), lambda i: (i, 0)),
        ],
        out_shape=[
            jax.ShapeDtypeStruct((N, H2), jnp.float32),
            jax.ShapeDtypeStruct((N, H2), jnp.float32),
            jax.ShapeDtypeStruct((N, H2), jnp.bfloat16),
        ],
        compiler_params=pltpu.CompilerParams(
            dimension_semantics=("arbitrary",)),
    )(adj, m, wcat, eps_u)

    preds_u, preds_a = pl.pallas_call(
        _k4_body,
        grid=(ND, ND),
        in_specs=[
            pl.BlockSpec((BD, H2), lambda i, j: (i, 0)),
            pl.BlockSpec((BD, H2), lambda i, j: (j, 0)),
            pl.BlockSpec((F, H2), lambda i, j: (0, 0)),
        ],
        out_specs=[
            pl.BlockSpec((BD, BD), lambda i, j: (i, j)),
            pl.BlockSpec((BD, F), lambda i, j: (i, 0)),
        ],
        out_shape=[
            jax.ShapeDtypeStruct((N, N), jnp.float32),
            jax.ShapeDtypeStruct((N, F), jnp.float32),
        ],
        compiler_params=pltpu.CompilerParams(
            dimension_semantics=("parallel", "arbitrary")),
    )(zu_bf, zu_bf, za_bf)

    return (preds_u, preds_a, zu_mean, zu_log_std, za_mean, za_log_std)
